# baseline (speedup 1.0000x reference)
import functools

import jax
import jax.numpy as jnp
from jax import lax
from jax.experimental import pallas as pl
from jax.experimental.pallas import tpu as pltpu
from jax.experimental.pallas import tpu_sc as plsc

_NC = 2
_NS = 16
_NW = _NC * _NS
_CHUNK = 32


@functools.partial(jax.jit, static_argnames=("d_model",))
def _sc_embed(idx, W_E, d_model):
    n_per = idx.shape[1]
    total = _NW * n_per
    sizes = [_CHUNK] * (n_per // _CHUNK)
    if n_per % _CHUNK:
        sizes.append(n_per % _CHUNK)
    offs = [sum(sizes[:j]) for j in range(len(sizes))]
    n = len(sizes)
    mesh = plsc.VectorSubcoreMesh(core_axis_name="c", subcore_axis_name="s")

    nbuf = 3

    @functools.partial(
        pl.kernel,
        out_type=jax.ShapeDtypeStruct((total, d_model), jnp.float32),
        mesh=mesh,
        scratch_types=[
            pltpu.VMEM((n_per,), jnp.int32),
            pltpu.VMEM((nbuf, _CHUNK, d_model), jnp.float32),
            [pltpu.SemaphoreType.DMA] * nbuf,
            [pltpu.SemaphoreType.DMA] * nbuf,
        ],
    )
    def k(idx_hbm, table_hbm, out_hbm, idx_v, bufs, gsems, ssems):
        wid = lax.axis_index("s") * _NC + lax.axis_index("c")
        base = wid * n_per
        pltpu.sync_copy(idx_hbm.at[wid], idx_v)

        def gather(j):
            b = j % nbuf
            return pltpu.make_async_copy(
                table_hbm.at[idx_v.at[pl.ds(offs[j], sizes[j])]],
                bufs.at[b].at[pl.ds(0, sizes[j])],
                gsems[b],
            )

        def store(j):
            b = j % nbuf
            return pltpu.make_async_copy(
                bufs.at[b].at[pl.ds(0, sizes[j])],
                out_hbm.at[pl.ds(base + offs[j], sizes[j])],
                ssems[b],
            )

        for j in range(min(nbuf, n)):
            gather(j).start()
        for j in range(n):
            gather(j).wait()
            store(j).start()
            if j + nbuf < n:
                store(j).wait()
                gather(j + nbuf).start()
        for j in range(max(0, n - nbuf), n):
            store(j).wait()

    return k(idx, W_E)


def kernel(tokens, W_E):
    B, T = tokens.shape
    V, D = W_E.shape
    idx = tokens.reshape(_NW, (B * T) // _NW).astype(jnp.int32)
    out = _sc_embed(idx, W_E, D)
    return out.reshape(B, T, D)
